# baseline (speedup 1.0000x reference)
import functools

import jax
import jax.numpy as jnp
from jax import lax
from jax.experimental import pallas as pl
from jax.experimental.pallas import tpu as pltpu
from jax.experimental.pallas import tpu_sc as plsc

_NC = 2
_NS = 16
_GRP = 128


def _make_spmm(Np, G, GPW, with_count):
    CH = 32 if with_count else 16
    NPT = Np // _NS
    mesh = plsc.VectorSubcoreMesh(core_axis_name="c", subcore_axis_name="s")

    NB = 4

    def body(tbl_hbm, ei_hbm, w_hbm, zeros_hbm, out_hbm,
             src_v, dst_v, w_v, rows_v, buf_v,
             acc_sh, gsem_a, gsem_b, gsem_c, gsem_d, ssem_a, ssem_b):
        c = lax.axis_index("c")
        s = lax.axis_index("s")
        wid = s
        gsem = (gsem_a, gsem_b, gsem_c, gsem_d)
        ssem = (ssem_a, ssem_b)
        g0 = wid * GPW

        @pl.when(c == 0)
        def _prologue():
            pltpu.sync_copy(ei_hbm.at[0, pl.ds(g0, GPW)], src_v)
            for b in range(NB):
                pltpu.async_copy(tbl_hbm.at[src_v.at[b]], rows_v.at[b], gsem[b])
            pltpu.sync_copy(ei_hbm.at[1, pl.ds(g0, GPW)], dst_v)
            pltpu.sync_copy(w_hbm.at[pl.ds(g0, GPW)], w_v)
        r0 = s * NPT
        pltpu.sync_copy(zeros_hbm.at[pl.ds(r0, NPT)], acc_sh.at[pl.ds(r0, NPT)])

        ones16 = jnp.full((16,), 1.0, jnp.float32)
        zeros16v = jnp.zeros((16,), jnp.float32)
        if with_count:
            for b in range(2):
                for j in range(_GRP):
                    buf_v[b, j, 16:32] = ones16
        plsc.subcore_barrier()

        def ring(i4, carry):
            for b in range(NB):
                g = i4 * NB + b
                sb = b % 2
                pltpu.make_async_copy(
                    tbl_hbm.at[src_v.at[g]], rows_v.at[b], gsem[b]).wait()

                @pl.when(g >= 2)
                def _wait_scatter():
                    pltpu.make_async_copy(
                        buf_v.at[sb], acc_sh.at[dst_v.at[g]], ssem[sb]).wait()

                if with_count:
                    @pl.when(g0 + g >= G)
                    def _pad_zeros():
                        for j in range(_GRP):
                            buf_v[sb, j, 16:32] = zeros16v

                for j16 in range(_GRP // 16):
                    wv = w_v[g, pl.ds(j16 * 16, 16)]
                    for jj in range(16):
                        j = j16 * 16 + jj
                        scaled = rows_v[b, j, :] * wv[jj]
                        if with_count:
                            buf_v[sb, j, 0:16] = scaled
                        else:
                            buf_v[sb, j, :] = scaled

                @pl.when(g + NB < GPW)
                def _next_gather():
                    pltpu.async_copy(
                        tbl_hbm.at[src_v.at[g + NB]], rows_v.at[b], gsem[b])

                pltpu.async_copy(
                    buf_v.at[sb], acc_sh.at[dst_v.at[g]], ssem[sb], add=True)
            return carry

        @pl.when(c == 0)
        def _main():
            lax.fori_loop(0, GPW // NB, ring, 0)
            pltpu.make_async_copy(
                buf_v.at[0], acc_sh.at[dst_v.at[0]], ssem[0]).wait()
            pltpu.make_async_copy(
                buf_v.at[1], acc_sh.at[dst_v.at[1]], ssem[1]).wait()
        plsc.subcore_barrier()
        pltpu.sync_copy(acc_sh.at[pl.ds(r0, NPT)], out_hbm.at[c, pl.ds(r0, NPT)])

    return pl.kernel(
        body,
        mesh=mesh,
        compiler_params=pltpu.CompilerParams(use_tc_tiling_on_sc=False),
        out_type=jax.ShapeDtypeStruct((_NC, Np, CH), jnp.float32),
        scratch_types=[
            pltpu.VMEM((GPW, _GRP), jnp.int32),
            pltpu.VMEM((GPW, _GRP), jnp.int32),
            pltpu.VMEM((GPW, _GRP), jnp.float32),
            pltpu.VMEM((4, _GRP, 16), jnp.float32),
            pltpu.VMEM((2, _GRP, CH), jnp.float32),
            pltpu.VMEM_SHARED((Np, CH), jnp.float32),
            pltpu.SemaphoreType.DMA,
            pltpu.SemaphoreType.DMA,
            pltpu.SemaphoreType.DMA,
            pltpu.SemaphoreType.DMA,
            pltpu.SemaphoreType.DMA,
            pltpu.SemaphoreType.DMA,
        ],
    )


def _tc_a_body(x_ref, w1_ref, va_ref, xt_ref, xv_ref):
    xt = jnp.dot(x_ref[...], w1_ref[...], preferred_element_type=jnp.float32)
    xt_ref[...] = xt
    xv_ref[...] = jnp.dot(xt, va_ref[...], preferred_element_type=jnp.float32)


def _tc_b_body(xt1_ref, zc_ref, ua_ref, w1b_ref, h1_ref, xt2_ref):
    xt1 = xt1_ref[...]
    z1 = zc_ref[0, :, 0:16] + zc_ref[1, :, 0:16]
    cnt = jnp.maximum(zc_ref[0, :, 16:17] + zc_ref[1, :, 16:17], 1.0)
    agg = z1 / cnt
    upd = jnp.dot(xt1, ua_ref[...], preferred_element_type=jnp.float32) + agg
    m = jnp.mean(upd, axis=1, keepdims=True)
    v = jnp.mean((upd - m) ** 2, axis=1, keepdims=True)
    h1 = xt1 + jnp.maximum((upd - m) * lax.rsqrt(v + 1e-5), 0.0)
    h1_ref[...] = h1
    xt2_ref[...] = jnp.dot(h1, w1b_ref[...], preferred_element_type=jnp.float32)


def _tc_c_body(xt2_ref, z2p_ref, zc_ref, ub_ref, w1b_ref, vb_ref,
               out_ref):
    xt2 = xt2_ref[...]
    z2 = z2p_ref[0] + z2p_ref[1]
    cnt = jnp.maximum(zc_ref[0, :, 16:17] + zc_ref[1, :, 16:17], 1.0)
    wv = jnp.dot(w1b_ref[...], vb_ref[...], preferred_element_type=jnp.float32)
    agg = jnp.dot(z2, wv, preferred_element_type=jnp.float32) / cnt
    upd = jnp.dot(xt2, ub_ref[...], preferred_element_type=jnp.float32) + agg
    m = jnp.mean(upd, axis=1, keepdims=True)
    v = jnp.mean((upd - m) ** 2, axis=1, keepdims=True)
    out_ref[...] = xt2 + jnp.maximum((upd - m) * lax.rsqrt(v + 1e-5), 0.0)


def kernel(X, n_id, edge_index, edge_weight, w1a, w2a, ua, va, w1b, w2b, ub, vb):
    del n_id
    N, Cin = X.shape[1], X.shape[2]
    H = w1a.shape[1]
    Cout = w1b.shape[1]
    E = edge_weight.shape[0]
    x2d = X[0]

    G = -(-E // _GRP)
    GPW = -(--(-G // _NS) // 8) * 8
    Gp = GPW * _NC * _NS
    pad = Gp * _GRP - E
    Np = -(-N // (8 * _NS)) * (8 * _NS)
    ei3 = jnp.pad(edge_index, ((0, 0), (0, pad))).reshape(2, Gp, _GRP)
    wpad = jnp.pad(edge_weight, (0, pad)).reshape(Gp, _GRP)
    zeros32 = jnp.zeros((Np, 32), jnp.float32)
    zeros16 = jnp.zeros((Np, 16), jnp.float32)

    va_eff = va * w2a[0][None, :]
    vb_eff = vb * w2b[0][None, :]

    BN = 1000
    nb = N // BN

    xt1, xv1 = pl.pallas_call(
        _tc_a_body,
        grid=(nb,),
        in_specs=[
            pl.BlockSpec((BN, Cin), lambda i: (i, 0)),
            pl.BlockSpec((Cin, H), lambda i: (0, 0)),
            pl.BlockSpec((H, H), lambda i: (0, 0)),
        ],
        out_specs=[
            pl.BlockSpec((BN, H), lambda i: (i, 0)),
            pl.BlockSpec((BN, H), lambda i: (i, 0)),
        ],
        out_shape=[
            jax.ShapeDtypeStruct((N, H), jnp.float32),
            jax.ShapeDtypeStruct((Np, H), jnp.float32),
        ],
    )(x2d, w1a, va_eff)

    zc = _make_spmm(Np, G, GPW, with_count=True)(xv1, ei3, wpad, zeros32)

    h1, xt2 = pl.pallas_call(
        _tc_b_body,
        grid=(nb,),
        in_specs=[
            pl.BlockSpec((BN, H), lambda i: (i, 0)),
            pl.BlockSpec((_NC, BN, 32), lambda i: (0, i, 0)),
            pl.BlockSpec((H, H), lambda i: (0, 0)),
            pl.BlockSpec((H, Cout), lambda i: (0, 0)),
        ],
        out_specs=[
            pl.BlockSpec((BN, H), lambda i: (i, 0)),
            pl.BlockSpec((BN, Cout), lambda i: (i, 0)),
        ],
        out_shape=[
            jax.ShapeDtypeStruct((Np, H), jnp.float32),
            jax.ShapeDtypeStruct((N, Cout), jnp.float32),
        ],
    )(xt1, zc, ua, w1b)

    z2p = _make_spmm(Np, G, GPW, with_count=False)(h1, ei3, wpad, zeros16)

    out = pl.pallas_call(
        _tc_c_body,
        grid=(nb,),
        in_specs=[
            pl.BlockSpec((BN, Cout), lambda i: (i, 0)),
            pl.BlockSpec((_NC, BN, 16), lambda i: (0, i, 0)),
            pl.BlockSpec((_NC, BN, 32), lambda i: (0, i, 0)),
            pl.BlockSpec((Cout, Cout), lambda i: (0, 0)),
            pl.BlockSpec((H, Cout), lambda i: (0, 0)),
            pl.BlockSpec((Cout, Cout), lambda i: (0, 0)),
        ],
        out_specs=pl.BlockSpec((BN, Cout), lambda i: (i, 0)),
        out_shape=jax.ShapeDtypeStruct((N, Cout), jnp.float32),
    )(xt2, z2p, zc, ub, w1b, vb_eff)

    return out[None]


# speedup vs baseline: 1.1953x; 1.1953x over previous
import functools

import jax
import jax.numpy as jnp
from jax import lax
from jax.experimental import pallas as pl
from jax.experimental.pallas import tpu as pltpu
from jax.experimental.pallas import tpu_sc as plsc

_NC = 2
_NS = 16
_GRP = 128


def _make_spmm(Np, G, GPW, with_count):
    CH = 32 if with_count else 16
    NPT = Np // _NS
    mesh = plsc.VectorSubcoreMesh(core_axis_name="c", subcore_axis_name="s")

    NB = 8

    def body(tbl_hbm, ei_hbm, w_hbm, zeros_hbm, out_hbm,
             src_v, dst_v, w_v, rows_v, buf_v,
             acc_sh, gsem_a, gsem_b, gsem_c, gsem_d,
             gsem_e, gsem_f, gsem_g, gsem_h, ssem_a, ssem_b):
        c = lax.axis_index("c")
        s = lax.axis_index("s")
        wid = c * _NS + s
        gsem = (gsem_a, gsem_b, gsem_c, gsem_d, gsem_e, gsem_f, gsem_g, gsem_h)
        ssem = (ssem_a, ssem_b)
        g0 = wid * GPW
        pltpu.sync_copy(ei_hbm.at[0, pl.ds(g0, GPW)], src_v)
        for b in range(NB):
            pltpu.async_copy(tbl_hbm.at[src_v.at[b]], rows_v.at[b], gsem[b])
        pltpu.sync_copy(ei_hbm.at[1, pl.ds(g0, GPW)], dst_v)
        pltpu.sync_copy(w_hbm.at[pl.ds(g0, GPW)], w_v)
        r0 = s * NPT
        pltpu.sync_copy(zeros_hbm.at[pl.ds(r0, NPT)], acc_sh.at[pl.ds(r0, NPT)])

        ones16 = jnp.full((16,), 1.0, jnp.float32)
        zeros16v = jnp.zeros((16,), jnp.float32)
        if with_count:
            for b in range(2):
                for j in range(_GRP):
                    buf_v[b, j, 16:32] = ones16
        plsc.subcore_barrier()

        def ring(i4, carry):
            for b in range(NB):
                g = i4 * NB + b
                sb = b % 2
                pltpu.make_async_copy(
                    tbl_hbm.at[src_v.at[g]], rows_v.at[b], gsem[b]).wait()

                @pl.when(g >= 2)
                def _wait_scatter():
                    pltpu.make_async_copy(
                        buf_v.at[sb], acc_sh.at[dst_v.at[g]], ssem[sb]).wait()

                if with_count:
                    @pl.when(g0 + g >= G)
                    def _pad_zeros():
                        for j in range(_GRP):
                            buf_v[sb, j, 16:32] = zeros16v

                for j16 in range(_GRP // 16):
                    wv = w_v[g, pl.ds(j16 * 16, 16)]
                    for jj in range(16):
                        j = j16 * 16 + jj
                        scaled = rows_v[b, j, :] * wv[jj]
                        if with_count:
                            buf_v[sb, j, 0:16] = scaled
                        else:
                            buf_v[sb, j, :] = scaled

                @pl.when(g + NB < GPW)
                def _next_gather():
                    pltpu.async_copy(
                        tbl_hbm.at[src_v.at[g + NB]], rows_v.at[b], gsem[b])

                pltpu.async_copy(
                    buf_v.at[sb], acc_sh.at[dst_v.at[g]], ssem[sb], add=True)
            return carry

        lax.fori_loop(0, GPW // NB, ring, 0)
        pltpu.make_async_copy(
            buf_v.at[0], acc_sh.at[dst_v.at[0]], ssem[0]).wait()
        pltpu.make_async_copy(
            buf_v.at[1], acc_sh.at[dst_v.at[1]], ssem[1]).wait()
        plsc.subcore_barrier()
        pltpu.sync_copy(acc_sh.at[pl.ds(r0, NPT)], out_hbm.at[c, pl.ds(r0, NPT)])

    return pl.kernel(
        body,
        mesh=mesh,
        compiler_params=pltpu.CompilerParams(use_tc_tiling_on_sc=False),
        out_type=jax.ShapeDtypeStruct((_NC, Np, CH), jnp.float32),
        scratch_types=[
            pltpu.VMEM((GPW, _GRP), jnp.int32),
            pltpu.VMEM((GPW, _GRP), jnp.int32),
            pltpu.VMEM((GPW, _GRP), jnp.float32),
            pltpu.VMEM((8, _GRP, 16), jnp.float32),
            pltpu.VMEM((2, _GRP, CH), jnp.float32),
            pltpu.VMEM_SHARED((Np, CH), jnp.float32),
            pltpu.SemaphoreType.DMA,
            pltpu.SemaphoreType.DMA,
            pltpu.SemaphoreType.DMA,
            pltpu.SemaphoreType.DMA,
            pltpu.SemaphoreType.DMA,
            pltpu.SemaphoreType.DMA,
            pltpu.SemaphoreType.DMA,
            pltpu.SemaphoreType.DMA,
            pltpu.SemaphoreType.DMA,
            pltpu.SemaphoreType.DMA,
        ],
    )


def _tc_a_body(x_ref, w1_ref, va_ref, xt_ref, xv_ref):
    xt = jnp.dot(x_ref[...], w1_ref[...], preferred_element_type=jnp.float32)
    xt_ref[...] = xt
    xv_ref[...] = jnp.dot(xt, va_ref[...], preferred_element_type=jnp.float32)


def _tc_b_body(xt1_ref, zc_ref, ua_ref, w1b_ref, h1_ref, xt2_ref):
    xt1 = xt1_ref[...]
    z1 = zc_ref[0, :, 0:16] + zc_ref[1, :, 0:16]
    cnt = jnp.maximum(zc_ref[0, :, 16:17] + zc_ref[1, :, 16:17], 1.0)
    agg = z1 / cnt
    upd = jnp.dot(xt1, ua_ref[...], preferred_element_type=jnp.float32) + agg
    m = jnp.mean(upd, axis=1, keepdims=True)
    v = jnp.mean((upd - m) ** 2, axis=1, keepdims=True)
    h1 = xt1 + jnp.maximum((upd - m) * lax.rsqrt(v + 1e-5), 0.0)
    h1_ref[...] = h1
    xt2_ref[...] = jnp.dot(h1, w1b_ref[...], preferred_element_type=jnp.float32)


def _tc_c_body(xt2_ref, z2p_ref, zc_ref, ub_ref, w1b_ref, vb_ref,
               out_ref):
    xt2 = xt2_ref[...]
    z2 = z2p_ref[0] + z2p_ref[1]
    cnt = jnp.maximum(zc_ref[0, :, 16:17] + zc_ref[1, :, 16:17], 1.0)
    wv = jnp.dot(w1b_ref[...], vb_ref[...], preferred_element_type=jnp.float32)
    agg = jnp.dot(z2, wv, preferred_element_type=jnp.float32) / cnt
    upd = jnp.dot(xt2, ub_ref[...], preferred_element_type=jnp.float32) + agg
    m = jnp.mean(upd, axis=1, keepdims=True)
    v = jnp.mean((upd - m) ** 2, axis=1, keepdims=True)
    out_ref[...] = xt2 + jnp.maximum((upd - m) * lax.rsqrt(v + 1e-5), 0.0)


def kernel(X, n_id, edge_index, edge_weight, w1a, w2a, ua, va, w1b, w2b, ub, vb):
    del n_id
    N, Cin = X.shape[1], X.shape[2]
    H = w1a.shape[1]
    Cout = w1b.shape[1]
    E = edge_weight.shape[0]
    x2d = X[0]

    G = -(-E // _GRP)
    GPW = -(--(-G // (_NC * _NS)) // 8) * 8
    Gp = GPW * _NC * _NS
    pad = Gp * _GRP - E
    Np = -(-N // (8 * _NS)) * (8 * _NS)
    ei3 = jnp.pad(edge_index, ((0, 0), (0, pad))).reshape(2, Gp, _GRP)
    wpad = jnp.pad(edge_weight, (0, pad)).reshape(Gp, _GRP)
    zeros32 = jnp.zeros((Np, 32), jnp.float32)
    zeros16 = jnp.zeros((Np, 16), jnp.float32)

    va_eff = va * w2a[0][None, :]
    vb_eff = vb * w2b[0][None, :]

    BN = 1000
    nb = N // BN

    xt1, xv1 = pl.pallas_call(
        _tc_a_body,
        grid=(nb,),
        in_specs=[
            pl.BlockSpec((BN, Cin), lambda i: (i, 0)),
            pl.BlockSpec((Cin, H), lambda i: (0, 0)),
            pl.BlockSpec((H, H), lambda i: (0, 0)),
        ],
        out_specs=[
            pl.BlockSpec((BN, H), lambda i: (i, 0)),
            pl.BlockSpec((BN, H), lambda i: (i, 0)),
        ],
        out_shape=[
            jax.ShapeDtypeStruct((N, H), jnp.float32),
            jax.ShapeDtypeStruct((Np, H), jnp.float32),
        ],
    )(x2d, w1a, va_eff)

    zc = _make_spmm(Np, G, GPW, with_count=True)(xv1, ei3, wpad, zeros32)

    h1, xt2 = pl.pallas_call(
        _tc_b_body,
        grid=(nb,),
        in_specs=[
            pl.BlockSpec((BN, H), lambda i: (i, 0)),
            pl.BlockSpec((_NC, BN, 32), lambda i: (0, i, 0)),
            pl.BlockSpec((H, H), lambda i: (0, 0)),
            pl.BlockSpec((H, Cout), lambda i: (0, 0)),
        ],
        out_specs=[
            pl.BlockSpec((BN, H), lambda i: (i, 0)),
            pl.BlockSpec((BN, Cout), lambda i: (i, 0)),
        ],
        out_shape=[
            jax.ShapeDtypeStruct((Np, H), jnp.float32),
            jax.ShapeDtypeStruct((N, Cout), jnp.float32),
        ],
    )(xt1, zc, ua, w1b)

    z2p = _make_spmm(Np, G, GPW, with_count=False)(h1, ei3, wpad, zeros16)

    out = pl.pallas_call(
        _tc_c_body,
        grid=(nb,),
        in_specs=[
            pl.BlockSpec((BN, Cout), lambda i: (i, 0)),
            pl.BlockSpec((_NC, BN, 16), lambda i: (0, i, 0)),
            pl.BlockSpec((_NC, BN, 32), lambda i: (0, i, 0)),
            pl.BlockSpec((Cout, Cout), lambda i: (0, 0)),
            pl.BlockSpec((H, Cout), lambda i: (0, 0)),
            pl.BlockSpec((Cout, Cout), lambda i: (0, 0)),
        ],
        out_specs=pl.BlockSpec((BN, Cout), lambda i: (i, 0)),
        out_shape=jax.ShapeDtypeStruct((N, Cout), jnp.float32),
    )(xt2, z2p, zc, ub, w1b, vb_eff)

    return out[None]


# speedup vs baseline: 1.5498x; 1.2966x over previous
import functools

import jax
import jax.numpy as jnp
from jax import lax
from jax.experimental import pallas as pl
from jax.experimental.pallas import tpu as pltpu
from jax.experimental.pallas import tpu_sc as plsc

_NC = 2
_NS = 16
_GRP = 128


def _make_spmm(Np, G, GPW, with_count):
    CH = 32 if with_count else 16
    NPT = Np // _NS
    mesh = plsc.VectorSubcoreMesh(core_axis_name="c", subcore_axis_name="s")

    NB = 4

    def body(tbl_hbm, ei_hbm, w_hbm, zeros_hbm, out_hbm,
             src_v, dst_v, w_v, rows_v, buf_v,
             tbl_sh, acc_sh, gsem_a, gsem_b, gsem_c, gsem_d, ssem_a, ssem_b):
        c = lax.axis_index("c")
        s = lax.axis_index("s")
        wid = c * _NS + s
        gsem = (gsem_a, gsem_b, gsem_c, gsem_d)
        ssem = (ssem_a, ssem_b)
        g0 = wid * GPW
        pltpu.sync_copy(ei_hbm.at[0, pl.ds(g0, GPW)], src_v)
        r0 = s * NPT
        pltpu.sync_copy(tbl_hbm.at[pl.ds(r0, NPT)], tbl_sh.at[pl.ds(r0, NPT)])
        pltpu.sync_copy(ei_hbm.at[1, pl.ds(g0, GPW)], dst_v)
        pltpu.sync_copy(w_hbm.at[pl.ds(g0, GPW)], w_v)
        pltpu.sync_copy(zeros_hbm.at[pl.ds(r0, NPT)], acc_sh.at[pl.ds(r0, NPT)])

        ones16 = jnp.full((16,), 1.0, jnp.float32)
        zeros16v = jnp.zeros((16,), jnp.float32)
        if with_count:
            for b in range(2):
                for j in range(_GRP):
                    buf_v[b, j, 16:32] = ones16
        plsc.subcore_barrier()
        for b in range(NB):
            pltpu.async_copy(tbl_sh.at[src_v.at[b]], rows_v.at[b], gsem[b])

        def ring(i4, carry):
            for b in range(NB):
                g = i4 * NB + b
                sb = b % 2
                pltpu.make_async_copy(
                    tbl_sh.at[src_v.at[g]], rows_v.at[b], gsem[b]).wait()

                @pl.when(g >= 2)
                def _wait_scatter():
                    pltpu.make_async_copy(
                        buf_v.at[sb], acc_sh.at[dst_v.at[g]], ssem[sb]).wait()

                if with_count:
                    @pl.when(g0 + g >= G)
                    def _pad_zeros():
                        for j in range(_GRP):
                            buf_v[sb, j, 16:32] = zeros16v

                for j16 in range(_GRP // 16):
                    wv = w_v[g, pl.ds(j16 * 16, 16)]
                    for jj in range(16):
                        j = j16 * 16 + jj
                        scaled = rows_v[b, j, :] * wv[jj]
                        if with_count:
                            buf_v[sb, j, 0:16] = scaled
                        else:
                            buf_v[sb, j, :] = scaled

                @pl.when(g + NB < GPW)
                def _next_gather():
                    pltpu.async_copy(
                        tbl_sh.at[src_v.at[g + NB]], rows_v.at[b], gsem[b])

                pltpu.async_copy(
                    buf_v.at[sb], acc_sh.at[dst_v.at[g]], ssem[sb], add=True)
            return carry

        lax.fori_loop(0, GPW // NB, ring, 0)
        pltpu.make_async_copy(
            buf_v.at[0], acc_sh.at[dst_v.at[0]], ssem[0]).wait()
        pltpu.make_async_copy(
            buf_v.at[1], acc_sh.at[dst_v.at[1]], ssem[1]).wait()
        plsc.subcore_barrier()
        pltpu.sync_copy(acc_sh.at[pl.ds(r0, NPT)], out_hbm.at[c, pl.ds(r0, NPT)])

    return pl.kernel(
        body,
        mesh=mesh,
        compiler_params=pltpu.CompilerParams(use_tc_tiling_on_sc=False),
        out_type=jax.ShapeDtypeStruct((_NC, Np, CH), jnp.float32),
        scratch_types=[
            pltpu.VMEM((GPW, _GRP), jnp.int32),
            pltpu.VMEM((GPW, _GRP), jnp.int32),
            pltpu.VMEM((GPW, _GRP), jnp.float32),
            pltpu.VMEM((4, _GRP, 16), jnp.float32),
            pltpu.VMEM((2, _GRP, CH), jnp.float32),
            pltpu.VMEM_SHARED((Np, 16), jnp.float32),
            pltpu.VMEM_SHARED((Np, CH), jnp.float32),
            pltpu.SemaphoreType.DMA,
            pltpu.SemaphoreType.DMA,
            pltpu.SemaphoreType.DMA,
            pltpu.SemaphoreType.DMA,
            pltpu.SemaphoreType.DMA,
            pltpu.SemaphoreType.DMA,
        ],
    )


def _tc_a_body(x_ref, w1_ref, va_ref, xt_ref, xv_ref):
    xt = jnp.dot(x_ref[...], w1_ref[...], preferred_element_type=jnp.float32)
    xt_ref[...] = xt
    xv_ref[...] = jnp.dot(xt, va_ref[...], preferred_element_type=jnp.float32)


def _tc_b_body(xt1_ref, zc_ref, ua_ref, w1b_ref, h1_ref, xt2_ref):
    xt1 = xt1_ref[...]
    z1 = zc_ref[0, :, 0:16] + zc_ref[1, :, 0:16]
    cnt = jnp.maximum(zc_ref[0, :, 16:17] + zc_ref[1, :, 16:17], 1.0)
    agg = z1 / cnt
    upd = jnp.dot(xt1, ua_ref[...], preferred_element_type=jnp.float32) + agg
    m = jnp.mean(upd, axis=1, keepdims=True)
    v = jnp.mean((upd - m) ** 2, axis=1, keepdims=True)
    h1 = xt1 + jnp.maximum((upd - m) * lax.rsqrt(v + 1e-5), 0.0)
    h1_ref[...] = h1
    xt2_ref[...] = jnp.dot(h1, w1b_ref[...], preferred_element_type=jnp.float32)


def _tc_c_body(xt2_ref, z2p_ref, zc_ref, ub_ref, w1b_ref, vb_ref,
               out_ref):
    xt2 = xt2_ref[...]
    z2 = z2p_ref[0] + z2p_ref[1]
    cnt = jnp.maximum(zc_ref[0, :, 16:17] + zc_ref[1, :, 16:17], 1.0)
    wv = jnp.dot(w1b_ref[...], vb_ref[...], preferred_element_type=jnp.float32)
    agg = jnp.dot(z2, wv, preferred_element_type=jnp.float32) / cnt
    upd = jnp.dot(xt2, ub_ref[...], preferred_element_type=jnp.float32) + agg
    m = jnp.mean(upd, axis=1, keepdims=True)
    v = jnp.mean((upd - m) ** 2, axis=1, keepdims=True)
    out_ref[...] = xt2 + jnp.maximum((upd - m) * lax.rsqrt(v + 1e-5), 0.0)


def kernel(X, n_id, edge_index, edge_weight, w1a, w2a, ua, va, w1b, w2b, ub, vb):
    del n_id
    N, Cin = X.shape[1], X.shape[2]
    H = w1a.shape[1]
    Cout = w1b.shape[1]
    E = edge_weight.shape[0]
    x2d = X[0]

    G = -(-E // _GRP)
    GPW = -(--(-G // (_NC * _NS)) // 8) * 8
    Gp = GPW * _NC * _NS
    pad = Gp * _GRP - E
    Np = -(-N // (8 * _NS)) * (8 * _NS)
    ei3 = jnp.pad(edge_index, ((0, 0), (0, pad))).reshape(2, Gp, _GRP)
    wpad = jnp.pad(edge_weight, (0, pad)).reshape(Gp, _GRP)
    zeros32 = jnp.zeros((Np, 32), jnp.float32)
    zeros16 = jnp.zeros((Np, 16), jnp.float32)

    va_eff = va * w2a[0][None, :]
    vb_eff = vb * w2b[0][None, :]

    BN = 1000
    nb = N // BN

    xt1, xv1 = pl.pallas_call(
        _tc_a_body,
        grid=(nb,),
        in_specs=[
            pl.BlockSpec((BN, Cin), lambda i: (i, 0)),
            pl.BlockSpec((Cin, H), lambda i: (0, 0)),
            pl.BlockSpec((H, H), lambda i: (0, 0)),
        ],
        out_specs=[
            pl.BlockSpec((BN, H), lambda i: (i, 0)),
            pl.BlockSpec((BN, H), lambda i: (i, 0)),
        ],
        out_shape=[
            jax.ShapeDtypeStruct((N, H), jnp.float32),
            jax.ShapeDtypeStruct((Np, H), jnp.float32),
        ],
    )(x2d, w1a, va_eff)

    zc = _make_spmm(Np, G, GPW, with_count=True)(xv1, ei3, wpad, zeros32)

    h1, xt2 = pl.pallas_call(
        _tc_b_body,
        grid=(nb,),
        in_specs=[
            pl.BlockSpec((BN, H), lambda i: (i, 0)),
            pl.BlockSpec((_NC, BN, 32), lambda i: (0, i, 0)),
            pl.BlockSpec((H, H), lambda i: (0, 0)),
            pl.BlockSpec((H, Cout), lambda i: (0, 0)),
        ],
        out_specs=[
            pl.BlockSpec((BN, H), lambda i: (i, 0)),
            pl.BlockSpec((BN, Cout), lambda i: (i, 0)),
        ],
        out_shape=[
            jax.ShapeDtypeStruct((Np, H), jnp.float32),
            jax.ShapeDtypeStruct((N, Cout), jnp.float32),
        ],
    )(xt1, zc, ua, w1b)

    z2p = _make_spmm(Np, G, GPW, with_count=False)(h1, ei3, wpad, zeros16)

    out = pl.pallas_call(
        _tc_c_body,
        grid=(nb,),
        in_specs=[
            pl.BlockSpec((BN, Cout), lambda i: (i, 0)),
            pl.BlockSpec((_NC, BN, 16), lambda i: (0, i, 0)),
            pl.BlockSpec((_NC, BN, 32), lambda i: (0, i, 0)),
            pl.BlockSpec((Cout, Cout), lambda i: (0, 0)),
            pl.BlockSpec((H, Cout), lambda i: (0, 0)),
            pl.BlockSpec((Cout, Cout), lambda i: (0, 0)),
        ],
        out_specs=pl.BlockSpec((BN, Cout), lambda i: (i, 0)),
        out_shape=jax.ShapeDtypeStruct((N, Cout), jnp.float32),
    )(xt2, z2p, zc, ub, w1b, vb_eff)

    return out[None]


# speedup vs baseline: 1.8090x; 1.1672x over previous
import functools

import jax
import jax.numpy as jnp
from jax import lax
from jax.experimental import pallas as pl
from jax.experimental.pallas import tpu as pltpu
from jax.experimental.pallas import tpu_sc as plsc

_NC = 2
_NS = 16
_GRP = 128


def _make_spmm(Np, G, GPW, with_count):
    CH = 32 if with_count else 16
    NPT = Np // _NS
    mesh = plsc.VectorSubcoreMesh(core_axis_name="c", subcore_axis_name="s")

    NB = 4

    def body(tbl_hbm, ei_hbm, w_hbm, zeros_hbm, out_hbm,
             src_v, dst_v, w_v, rows_v, buf_v,
             tbl_sh, acc_sh, gsem_a, gsem_b, gsem_c, gsem_d, ssem_a, ssem_b):
        c = lax.axis_index("c")
        s = lax.axis_index("s")
        wid = c * _NS + s
        gsem = (gsem_a, gsem_b, gsem_c, gsem_d)
        ssem = (ssem_a, ssem_b)
        g0 = wid * GPW
        pltpu.sync_copy(ei_hbm.at[0, pl.ds(g0, GPW)], src_v)
        r0 = s * NPT
        pltpu.sync_copy(tbl_hbm.at[pl.ds(r0, NPT)], tbl_sh.at[pl.ds(r0, NPT)])
        pltpu.sync_copy(ei_hbm.at[1, pl.ds(g0, GPW)], dst_v)
        pltpu.sync_copy(w_hbm.at[pl.ds(g0, GPW)], w_v)
        pltpu.sync_copy(zeros_hbm.at[pl.ds(r0, NPT)], acc_sh.at[pl.ds(r0, NPT)])

        ones16 = jnp.full((16,), 1.0, jnp.float32)
        zeros16v = jnp.zeros((16,), jnp.float32)
        if with_count:
            for b in range(2):
                for j in range(_GRP):
                    buf_v[b, j, 16:32] = ones16
        plsc.subcore_barrier()
        for b in range(NB):
            pltpu.async_copy(tbl_sh.at[src_v.at[b]], rows_v.at[b], gsem[b])

        def ring(i4, carry):
            for b in range(NB):
                g = i4 * NB + b
                sb = b % 2
                pltpu.make_async_copy(
                    tbl_sh.at[src_v.at[g]], rows_v.at[b], gsem[b]).wait()

                @pl.when(g >= 2)
                def _wait_scatter():
                    pltpu.make_async_copy(
                        buf_v.at[sb], acc_sh.at[dst_v.at[g]], ssem[sb]).wait()

                if with_count:
                    @pl.when(g0 + g >= G)
                    def _pad_zeros():
                        for j in range(_GRP):
                            buf_v[sb, j, 16:32] = zeros16v

                for j16 in range(_GRP // 16):
                    wv = w_v[g, pl.ds(j16 * 16, 16)]
                    for jj in range(16):
                        j = j16 * 16 + jj
                        scaled = rows_v[b, j, :] * wv[jj]
                        if with_count:
                            buf_v[sb, j, 0:16] = scaled
                        else:
                            buf_v[sb, j, :] = scaled

                @pl.when(g + NB < GPW)
                def _next_gather():
                    pltpu.async_copy(
                        tbl_sh.at[src_v.at[g + NB]], rows_v.at[b], gsem[b])

                pltpu.async_copy(
                    buf_v.at[sb], acc_sh.at[dst_v.at[g]], ssem[sb], add=True)
            return carry

        lax.fori_loop(0, GPW // NB, ring, 0)
        pltpu.make_async_copy(
            buf_v.at[0], acc_sh.at[dst_v.at[0]], ssem[0]).wait()
        pltpu.make_async_copy(
            buf_v.at[1], acc_sh.at[dst_v.at[1]], ssem[1]).wait()
        plsc.subcore_barrier()
        pltpu.sync_copy(acc_sh.at[pl.ds(r0, NPT)], out_hbm.at[c, pl.ds(r0, NPT)])

    return pl.kernel(
        body,
        mesh=mesh,
        compiler_params=pltpu.CompilerParams(use_tc_tiling_on_sc=False),
        out_type=jax.ShapeDtypeStruct((_NC, Np, CH), jnp.float32),
        scratch_types=[
            pltpu.VMEM((GPW, _GRP), jnp.int32),
            pltpu.VMEM((GPW, _GRP), jnp.int32),
            pltpu.VMEM((GPW, _GRP), jnp.float32),
            pltpu.VMEM((4, _GRP, 16), jnp.float32),
            pltpu.VMEM((2, _GRP, CH), jnp.float32),
            pltpu.VMEM_SHARED((Np, 16), jnp.float32),
            pltpu.VMEM_SHARED((Np, CH), jnp.float32),
            pltpu.SemaphoreType.DMA,
            pltpu.SemaphoreType.DMA,
            pltpu.SemaphoreType.DMA,
            pltpu.SemaphoreType.DMA,
            pltpu.SemaphoreType.DMA,
            pltpu.SemaphoreType.DMA,
        ],
    )


def _tc_a_body(x_ref, w1_ref, va_ref, xt_ref, xv_ref):
    w1 = w1_ref[...]
    va = va_ref[...]
    for k in range(8):
        xt = jnp.dot(x_ref[:, k, :], w1, preferred_element_type=jnp.float32)
        xt_ref[:, 16 * k:16 * k + 16] = xt
        xv_ref[:, 16 * k:16 * k + 16] = jnp.dot(
            xt, va, preferred_element_type=jnp.float32)


def _tc_b_body(xt1_ref, zc_ref, ua_ref, w1b_ref, h1_ref, xt2_ref):
    ua = ua_ref[...]
    w1b = w1b_ref[...]
    for k in range(8):
        xt1 = xt1_ref[:, 16 * k:16 * k + 16]
        z1 = zc_ref[0, :, 32 * k:32 * k + 16] + zc_ref[1, :, 32 * k:32 * k + 16]
        cnt = jnp.maximum(zc_ref[0, :, 32 * k + 16:32 * k + 17]
                          + zc_ref[1, :, 32 * k + 16:32 * k + 17], 1.0)
        agg = z1 / cnt
        upd = jnp.dot(xt1, ua, preferred_element_type=jnp.float32) + agg
        m = jnp.mean(upd, axis=1, keepdims=True)
        v = jnp.mean((upd - m) ** 2, axis=1, keepdims=True)
        h1 = xt1 + jnp.maximum((upd - m) * lax.rsqrt(v + 1e-5), 0.0)
        h1_ref[:, 16 * k:16 * k + 16] = h1
        xt2_ref[:, k, :] = jnp.dot(h1, w1b, preferred_element_type=jnp.float32)


def _tc_c_body(xt2_ref, z2p_ref, zc_ref, ub_ref, w1b_ref, vb_ref,
               out_ref):
    ub = ub_ref[...]
    wv = jnp.dot(w1b_ref[...], vb_ref[...], preferred_element_type=jnp.float32)
    for k in range(8):
        z2 = (z2p_ref[0, :, 16 * k:16 * k + 16]
              + z2p_ref[1, :, 16 * k:16 * k + 16])
        cnt = jnp.maximum(zc_ref[0, :, 32 * k + 16:32 * k + 17]
                          + zc_ref[1, :, 32 * k + 16:32 * k + 17], 1.0)
        agg = jnp.dot(z2, wv, preferred_element_type=jnp.float32) / cnt
        xt2 = xt2_ref[:, k, :]
        upd = jnp.dot(xt2, ub, preferred_element_type=jnp.float32) + agg
        m = jnp.mean(upd, axis=1, keepdims=True)
        v = jnp.mean((upd - m) ** 2, axis=1, keepdims=True)
        out_ref[:, k, :] = xt2 + jnp.maximum((upd - m) * lax.rsqrt(v + 1e-5),
                                             0.0)


def kernel(X, n_id, edge_index, edge_weight, w1a, w2a, ua, va, w1b, w2b, ub, vb):
    del n_id
    N, Cin = X.shape[1], X.shape[2]
    H = w1a.shape[1]
    Cout = w1b.shape[1]
    E = edge_weight.shape[0]
    x2d = X[0]

    G = -(-E // _GRP)
    GPW = -(--(-G // (_NC * _NS)) // 8) * 8
    Gp = GPW * _NC * _NS
    pad = Gp * _GRP - E
    NP8 = N // 8
    ei3 = jnp.pad(edge_index, ((0, 0), (0, pad))).reshape(2, Gp, _GRP)
    wpad = jnp.pad(edge_weight, (0, pad)).reshape(Gp, _GRP)
    zeros32 = jnp.zeros((N, 32), jnp.float32)
    zeros16 = jnp.zeros((N, 16), jnp.float32)

    va_eff = va * w2a[0][None, :]
    vb_eff = vb * w2b[0][None, :]

    x3 = X.reshape(NP8, 8, Cin)

    xt1p, xv1p = pl.pallas_call(
        _tc_a_body,
        grid=(1,),
        in_specs=[
            pl.BlockSpec((NP8, 8, Cin), lambda i: (0, 0, 0)),
            pl.BlockSpec((Cin, H), lambda i: (0, 0)),
            pl.BlockSpec((H, H), lambda i: (0, 0)),
        ],
        out_specs=[
            pl.BlockSpec((NP8, 8 * H), lambda i: (0, 0)),
            pl.BlockSpec((NP8, 8 * H), lambda i: (0, 0)),
        ],
        out_shape=[jax.ShapeDtypeStruct((NP8, 8 * H), jnp.float32)] * 2,
    )(x3, w1a, va_eff)

    zc = _make_spmm(N, G, GPW, with_count=True)(
        xv1p.reshape(N, H), ei3, wpad, zeros32)
    zc3 = zc.reshape(_NC, NP8, 8 * 32)

    h1p, xt23 = pl.pallas_call(
        _tc_b_body,
        grid=(1,),
        in_specs=[
            pl.BlockSpec((NP8, 8 * H), lambda i: (0, 0)),
            pl.BlockSpec((_NC, NP8, 8 * 32), lambda i: (0, 0, 0)),
            pl.BlockSpec((H, H), lambda i: (0, 0)),
            pl.BlockSpec((H, Cout), lambda i: (0, 0)),
        ],
        out_specs=[
            pl.BlockSpec((NP8, 8 * H), lambda i: (0, 0)),
            pl.BlockSpec((NP8, 8, Cout), lambda i: (0, 0, 0)),
        ],
        out_shape=[
            jax.ShapeDtypeStruct((NP8, 8 * H), jnp.float32),
            jax.ShapeDtypeStruct((NP8, 8, Cout), jnp.float32),
        ],
    )(xt1p, zc3, ua, w1b)

    z2p = _make_spmm(N, G, GPW, with_count=False)(
        h1p.reshape(N, H), ei3, wpad, zeros16)
    z23 = z2p.reshape(_NC, NP8, 8 * H)

    out3 = pl.pallas_call(
        _tc_c_body,
        grid=(1,),
        in_specs=[
            pl.BlockSpec((NP8, 8, Cout), lambda i: (0, 0, 0)),
            pl.BlockSpec((_NC, NP8, 8 * H), lambda i: (0, 0, 0)),
            pl.BlockSpec((_NC, NP8, 8 * 32), lambda i: (0, 0, 0)),
            pl.BlockSpec((Cout, Cout), lambda i: (0, 0)),
            pl.BlockSpec((H, Cout), lambda i: (0, 0)),
            pl.BlockSpec((Cout, Cout), lambda i: (0, 0)),
        ],
        out_specs=pl.BlockSpec((NP8, 8, Cout), lambda i: (0, 0, 0)),
        out_shape=jax.ShapeDtypeStruct((NP8, 8, Cout), jnp.float32),
    )(xt23, z23, zc3, ub, w1b, vb_eff)

    return out3.reshape(1, N, Cout)
